# Initial kernel scaffold; baseline (speedup 1.0000x reference)
import jax
import jax.numpy as jnp
from jax.experimental import pallas as pl


def kernel(indices, table):
    raise NotImplementedError("write your pallas kernel here")


# speedup vs baseline: 1.0928x; 1.0928x over previous
import functools

import jax
import jax.numpy as jnp
from jax import lax
from jax.experimental import pallas as pl
from jax.experimental.pallas import tpu as pltpu
from jax.experimental.pallas import tpu_sc as plsc

VOCAB = 1000000
EMBED_DIM = 32
BATCH = 16384
HIST = 50

_TOTAL = BATCH * HIST
_NC, _NS = 2, 16
_NW = _NC * _NS
_B_PER_W = _TOTAL // _NW
_CHUNK = 1024
_NSTEPS = _B_PER_W // _CHUNK


def _make_gather():
    mesh = plsc.VectorSubcoreMesh(core_axis_name="c", subcore_axis_name="s")

    @functools.partial(
        pl.kernel,
        mesh=mesh,
        out_type=jax.ShapeDtypeStruct((_TOTAL, EMBED_DIM), jnp.float32),
        scratch_types=[
            pltpu.VMEM((_CHUNK,), jnp.int32),
            pltpu.VMEM((_CHUNK, EMBED_DIM), jnp.float32),
            pltpu.SemaphoreType.DMA,
        ],
        compiler_params=pltpu.CompilerParams(use_tc_tiling_on_sc=False),
    )
    def gather_kernel(idx_hbm, table_hbm, out_hbm, idx_v, rows_v, sem):
        wid = lax.axis_index("s") * _NC + lax.axis_index("c")
        base = wid * _B_PER_W

        def step(i, carry):
            off = base + i * _CHUNK
            pltpu.sync_copy(idx_hbm.at[pl.ds(off, _CHUNK)], idx_v)
            pltpu.async_copy(table_hbm.at[idx_v], rows_v, sem).wait()
            pltpu.sync_copy(rows_v, out_hbm.at[pl.ds(off, _CHUNK)])
            return carry

        lax.fori_loop(0, _NSTEPS, step, 0)

    return gather_kernel


_gather = _make_gather()


@jax.jit
def kernel(indices, table):
    idx_flat = indices.reshape(_TOTAL).astype(jnp.int32)
    out = _gather(idx_flat, table)
    return out.reshape(BATCH, HIST, EMBED_DIM)


# speedup vs baseline: 1.1078x; 1.0138x over previous
import functools

import jax
import jax.numpy as jnp
from jax import lax
from jax.experimental import pallas as pl
from jax.experimental.pallas import tpu as pltpu
from jax.experimental.pallas import tpu_sc as plsc

VOCAB = 1000000
EMBED_DIM = 32
BATCH = 16384
HIST = 50

_TOTAL = BATCH * HIST
_NC, _NS = 2, 16
_NW = _NC * _NS
_B_PER_W = _TOTAL // _NW
_CHUNK = 800
_NB = 4
_NSTEPS = _B_PER_W // _CHUNK
_NGROUPS = _NSTEPS // _NB


def _make_gather():
    mesh = plsc.VectorSubcoreMesh(core_axis_name="c", subcore_axis_name="s")

    @functools.partial(
        pl.kernel,
        mesh=mesh,
        out_type=jax.ShapeDtypeStruct((_TOTAL, EMBED_DIM), jnp.float32),
        scratch_types=(
            [pltpu.VMEM((_B_PER_W,), jnp.int32)]
            + [pltpu.VMEM((_CHUNK, EMBED_DIM), jnp.float32) for _ in range(_NB)]
            + [pltpu.SemaphoreType.DMA for _ in range(2 * _NB)]
        ),
        compiler_params=pltpu.CompilerParams(use_tc_tiling_on_sc=False),
    )
    def gather_kernel(idx_hbm, table_hbm, out_hbm, idx_v, r0, r1, r2, r3,
                      g0, g1, g2, g3, s0, s1, s2, s3):
        rows = (r0, r1, r2, r3)
        gsem = (g0, g1, g2, g3)
        ssem = (s0, s1, s2, s3)
        wid = lax.axis_index("s") * _NC + lax.axis_index("c")
        base = wid * _B_PER_W

        pltpu.sync_copy(idx_hbm.at[pl.ds(base, _B_PER_W)], idx_v)

        def gather_start(b, i):
            pltpu.async_copy(
                table_hbm.at[idx_v.at[pl.ds(i * _CHUNK, _CHUNK)]],
                rows[b], gsem[b])

        def store_start(b, i):
            pltpu.async_copy(
                rows[b], out_hbm.at[pl.ds(base + i * _CHUNK, _CHUNK)],
                ssem[b])

        def gather_wait(b, i):
            pltpu.make_async_copy(
                table_hbm.at[idx_v.at[pl.ds(i * _CHUNK, _CHUNK)]],
                rows[b], gsem[b]).wait()

        def store_wait(b, i):
            pltpu.make_async_copy(
                rows[b], out_hbm.at[pl.ds(base + i * _CHUNK, _CHUNK)],
                ssem[b]).wait()

        for b in range(_NB):
            gather_start(b, b)

        def group(gi, carry):
            g = gi * _NB
            for b in range(_NB):
                gather_wait(b, g + b)
                store_start(b, g + b)
            for b in range(_NB):
                store_wait(b, g + b)

                @pl.when(gi < _NGROUPS - 1)
                def _():
                    gather_start(b, g + _NB + b)
            return carry

        lax.fori_loop(0, _NGROUPS, group, 0)

    return gather_kernel


_gather = _make_gather()


@jax.jit
def kernel(indices, table):
    idx_flat = indices.reshape(_TOTAL).astype(jnp.int32)
    out = _gather(idx_flat, table)
    return out.reshape(BATCH, HIST, EMBED_DIM)


# speedup vs baseline: 1.6308x; 1.4720x over previous
import functools

import jax
import jax.numpy as jnp
from jax import lax
from jax.experimental import pallas as pl
from jax.experimental.pallas import tpu as pltpu
from jax.experimental.pallas import tpu_sc as plsc

VOCAB = 1000000
EMBED_DIM = 32
BATCH = 16384
HIST = 50

_TOTAL = BATCH * HIST
_NC, _NS = 2, 16
_NW = _NC * _NS
_BW = BATCH // _NW
_NG = 2
_NT = 2


def _make_gather():
    mesh = plsc.VectorSubcoreMesh(core_axis_name="c", subcore_axis_name="s")

    @functools.partial(
        pl.kernel,
        mesh=mesh,
        out_type=jax.ShapeDtypeStruct((HIST, 4, BATCH * 8), jnp.float32),
        scratch_types=(
            [pltpu.VMEM((_BW * HIST,), jnp.int32),
             pltpu.VMEM((HIST * _BW,), jnp.int32)]
            + [pltpu.VMEM((_BW, EMBED_DIM), jnp.float32) for _ in range(_NG)]
            + [pltpu.VMEM((4 * 4096,), jnp.float32) for _ in range(_NT)]
            + [pltpu.SemaphoreType.DMA for _ in range(_NG + _NT)]
        ),
        compiler_params=pltpu.CompilerParams(use_tc_tiling_on_sc=False,
                                             needs_layout_passes=False),
    )
    def gather_kernel(idx_hbm, table_hbm, out_hbm, idx_v, idx_t,
                      ga, gb, ta, tb, sga, sgb, sta, stb):
        rows = (ga, gb)
        tbuf = (ta, tb)
        gsem = (sga, sgb)
        ssem = (sta, stb)
        wid = lax.axis_index("s") * _NC + lax.axis_index("c")
        base_j = wid * _BW * HIST
        pbase = wid * (_BW * 8)

        iota16 = lax.iota(jnp.int32, 16)
        bvecs = [iota16 + bb * 16 for bb in range(_BW // 16)]
        bvecs50 = [v * HIST for v in bvecs]

        pltpu.sync_copy(idx_hbm.at[pl.ds(base_j, _BW * HIST)], idx_v)

        def idx_tr(h, carry):
            for bb in range(_BW // 16):
                vals = plsc.load_gather(idx_v, [bvecs50[bb] + h])
                idx_t[pl.ds(h * _BW + bb * 16, 16)] = vals
            return carry

        lax.fori_loop(0, HIST, idx_tr, 0)

        def gather_start(b, h):
            pltpu.async_copy(table_hbm.at[idx_t.at[pl.ds(h * _BW, _BW)]],
                             rows[b], gsem[b])

        def gather_wait(b, h):
            pltpu.make_async_copy(
                table_hbm.at[idx_t.at[pl.ds(h * _BW, _BW)]],
                rows[b], gsem[b]).wait()

        def store_start(b, h):
            for dt in range(4):
                pltpu.async_copy(
                    tbuf[b].at[pl.ds(dt * 4096, 4096)],
                    out_hbm.at[h, dt, pl.ds(pbase, 4096)], ssem[b])

        def store_wait(b, h):
            for dt in range(4):
                pltpu.make_async_copy(
                    tbuf[b].at[pl.ds(dt * 4096, 4096)],
                    out_hbm.at[h, dt, pl.ds(pbase, 4096)], ssem[b]).wait()

        def transpose(bg, bt_):
            g = rows[bg]
            t = tbuf[bt_]

            def per_d(d, carry):
                toff = (d // 8) * 4096 + (d % 8) * 128
                dv = iota16 * 0 + d
                for btl in range(4):
                    for l in range(8):
                        vals = plsc.load_gather(g, [bvecs[btl * 8 + l], dv])
                        t[pl.ds(toff + btl * 1024 + l * 16, 16)] = vals
                return carry

            lax.fori_loop(0, EMBED_DIM, per_d, 0)

        for k in range(_NG):
            gather_start(k, k)

        def group(g6, carry):
            h0 = g6 * 6
            for k in range(6):
                h = h0 + k
                bg = k % _NG
                bt_ = k % _NT
                gather_wait(bg, h)

                @pl.when(h >= 2)
                def _():
                    store_wait(bt_, h - 2)

                transpose(bg, bt_)
                store_start(bt_, h)

                @pl.when(h + _NG < HIST)
                def _():
                    gather_start(bg, h + _NG)
            return carry

        lax.fori_loop(0, 8, group, 0)

        for h in (48, 49):
            bg = h % _NG
            bt_ = h % _NT
            gather_wait(bg, h)
            store_wait(bt_, h - 2)
            transpose(bg, bt_)
            store_start(bt_, h)
        store_wait(0, 48)
        store_wait(1, 49)

    return gather_kernel


_gather = _make_gather()


@jax.jit
def kernel(indices, table):
    idx_flat = indices.reshape(_TOTAL).astype(jnp.int32)
    out5 = _gather(idx_flat, table)
    out5 = out5.reshape(HIST, 4, BATCH // 128, 8, 128)
    return out5.transpose(2, 4, 0, 1, 3).reshape(BATCH, HIST, EMBED_DIM)


# speedup vs baseline: 2.5346x; 1.5542x over previous
import functools

import jax
import jax.numpy as jnp
from jax import lax
from jax.experimental import pallas as pl
from jax.experimental.pallas import tpu as pltpu
from jax.experimental.pallas import tpu_sc as plsc

VOCAB = 1000000
EMBED_DIM = 32
BATCH = 16384
HIST = 50

_TOTAL = BATCH * HIST
_NC, _NS = 2, 16
_NW = _NC * _NS
_BW = BATCH // _NW
_NG = 2
_NT = 2


def _make_gather():
    mesh = plsc.VectorSubcoreMesh(core_axis_name="c", subcore_axis_name="s")

    @functools.partial(
        pl.kernel,
        mesh=mesh,
        out_type=jax.ShapeDtypeStruct((HIST, 4, BATCH * 8), jnp.float32),
        scratch_types=(
            [pltpu.VMEM((_BW * HIST,), jnp.int32),
             pltpu.VMEM((HIST * _BW,), jnp.int32)]
            + [pltpu.VMEM((_BW, EMBED_DIM), jnp.float32) for _ in range(_NG)]
            + [pltpu.VMEM((4 * 4096,), jnp.float32) for _ in range(_NT)]
            + [pltpu.SemaphoreType.DMA for _ in range(_NG + _NT)]
        ),
        compiler_params=pltpu.CompilerParams(use_tc_tiling_on_sc=False,
                                             needs_layout_passes=False),
    )
    def gather_kernel(idx_hbm, table_hbm, out_hbm, idx_v, idx_t,
                      ga, gb, ta, tb, sga, sgb, sta, stb):
        rows = (ga, gb)
        tbuf = (ta, tb)
        gsem = (sga, sgb)
        ssem = (sta, stb)
        wid = lax.axis_index("s") * _NC + lax.axis_index("c")
        base_j = wid * _BW * HIST
        pbase = wid * (_BW * 8)

        iota16 = lax.iota(jnp.int32, 16)
        dskew = [(iota16 + k) & 15 for k in range(16)]
        dconst = [((d0 >> 3) << 12) + ((d0 & 7) << 7) + iota16
                  for d0 in dskew]

        pltpu.sync_copy(idx_hbm.at[pl.ds(base_j, _BW * HIST)], idx_v)

        def idx_tr(h, carry):
            for bb in range(_BW // 16):
                src = (iota16 + bb * 16) * HIST + h
                vals = plsc.load_gather(idx_v, [src])
                idx_t[pl.ds(h * _BW + bb * 16, 16)] = vals
            return carry

        lax.fori_loop(0, HIST, idx_tr, 0)

        def gather_start(b, h):
            pltpu.async_copy(table_hbm.at[idx_t.at[pl.ds(h * _BW, _BW)]],
                             rows[b], gsem[b])

        def gather_wait(b, h):
            pltpu.make_async_copy(
                table_hbm.at[idx_t.at[pl.ds(h * _BW, _BW)]],
                rows[b], gsem[b]).wait()

        def store_start(b, h):
            for dt in range(4):
                pltpu.async_copy(
                    tbuf[b].at[pl.ds(dt * 4096, 4096)],
                    out_hbm.at[h, dt, pl.ds(pbase, 4096)], ssem[b])

        def store_wait(b, h):
            for dt in range(4):
                pltpu.make_async_copy(
                    tbuf[b].at[pl.ds(dt * 4096, 4096)],
                    out_hbm.at[h, dt, pl.ds(pbase, 4096)], ssem[b]).wait()

        def transpose(bg, bt_):
            g = rows[bg]
            t = tbuf[bt_]

            def per_bb(i, carry):
                bv = iota16 + i * 16
                tb = (i // 8) * 1024 + (i % 8) * 16
                for dh in range(2):
                    for k in range(16):
                        vals = plsc.load_gather(g, [bv, dskew[k] + dh * 16])
                        plsc.store_scatter(
                            t, [dconst[k] + (tb + dh * 8192)], vals)
                return carry

            lax.fori_loop(0, _BW // 16, per_bb, 0)

        for k in range(_NG):
            gather_start(k, k)

        def group(g6, carry):
            h0 = g6 * 6
            for k in range(6):
                h = h0 + k
                bg = k % _NG
                bt_ = k % _NT
                gather_wait(bg, h)

                @pl.when(h >= 2)
                def _():
                    store_wait(bt_, h - 2)

                transpose(bg, bt_)
                store_start(bt_, h)

                @pl.when(h + _NG < HIST)
                def _():
                    gather_start(bg, h + _NG)
            return carry

        lax.fori_loop(0, 8, group, 0)

        for h in (48, 49):
            bg = h % _NG
            bt_ = h % _NT
            gather_wait(bg, h)
            store_wait(bt_, h - 2)
            transpose(bg, bt_)
            store_start(bt_, h)
        store_wait(0, 48)
        store_wait(1, 49)

    return gather_kernel


_gather = _make_gather()


@jax.jit
def kernel(indices, table):
    idx_flat = indices.reshape(_TOTAL).astype(jnp.int32)
    out5 = _gather(idx_flat, table)
    out5 = out5.reshape(HIST, 4, BATCH // 128, 8, 128)
    return out5.transpose(2, 4, 0, 1, 3).reshape(BATCH, HIST, EMBED_DIM)
